# baseline (speedup 1.0000x reference)
import functools

import jax
import jax.numpy as jnp
import numpy as np
from jax import lax
from jax.experimental import pallas as pl
from jax.experimental.pallas import tpu as pltpu
from jax.experimental.pallas import tpu_sc as plsc

NC = 2
NS = 16
NW = NC * NS

K = 2
NPH = 2


def _pool_sc(table, xp, dstb, zeros, *, B, LP, PD):
    spt = B // NW
    hspt = spt // NPH
    ng = hspt // K
    grows = LP
    gbytes_rows = K * grows

    mesh = plsc.VectorSubcoreMesh(core_axis_name="c", subcore_axis_name="s")

    @functools.partial(
        pl.kernel,
        mesh=mesh,
        compiler_params=pltpu.CompilerParams(use_tc_tiling_on_sc=True),
        out_type=jax.ShapeDtypeStruct((B, PD), jnp.float32),
        scratch_types=[
            pltpu.VMEM((hspt * LP,), jnp.int32),
            pltpu.VMEM((hspt * LP,), jnp.int32),
            pltpu.VMEM((K * LP, PD), jnp.float32),
            pltpu.VMEM((K * LP, PD), jnp.float32),
            pltpu.VMEM_SHARED((NS * (B // NW) + 8, PD), jnp.float32),
            pltpu.SemaphoreType.DMA,
            pltpu.SemaphoreType.DMA,
            pltpu.SemaphoreType.DMA,
            pltpu.SemaphoreType.DMA,
        ],
    )
    def pool(table_h, x_h, dstb_h, zeros_h, out_h, idx_all, dst_all, rows0,
             rows1, acc, sem0, sem1, ssem0, ssem1):
        c = lax.axis_index("c")
        s = lax.axis_index("s")
        wid = c * NS + s
        row0 = s * spt

        pltpu.sync_copy(zeros_h, acc.at[pl.ds(row0, spt)])

        rows = (rows0, rows1)
        sems = (sem0, sem1)
        ssems = (ssem0, ssem1)

        def fire(g, h):
            for k in range(K):
                j = g * K + k
                pltpu.async_copy(
                    table_h.at[idx_all.at[pl.ds(j * LP, LP)]],
                    rows[h].at[pl.ds(k * grows, grows)],
                    sems[h],
                )

        def drain(sem, buf):
            pltpu.make_async_copy(
                table_h.at[pl.ds(0, gbytes_rows)], buf, sem
            ).wait()

        for ph in range(NPH):
            pltpu.sync_copy(
                x_h.at[pl.ds((wid * spt + ph * hspt) * LP, hspt * LP)],
                idx_all)
            pltpu.sync_copy(
                dstb_h.at[pl.ds((s * spt + ph * hspt) * LP, hspt * LP)],
                dst_all)

            fire(0, 0)

            def outer(gg, carry):
                for hh in range(2):
                    g = gg * 2 + hh
                    nh = (hh + 1) % 2

                    @pl.when(g + 1 < ng)
                    def _():
                        @pl.when(g >= 1)
                        def _():
                            drain(ssems[nh], rows[nh])
                        fire(g + 1, nh)

                    drain(sems[hh], rows[hh])
                    for k in range(K):
                        j = g * K + k
                        pltpu.async_copy(
                            rows[hh].at[pl.ds(k * grows, grows)],
                            acc.at[dst_all.at[pl.ds(j * LP, LP)]],
                            ssems[hh],
                            add=True,
                        )
                return carry

            lax.fori_loop(0, ng // 2, outer, 0)
            drain(ssems[0], rows[0])
            drain(ssems[1], rows[1])

        pltpu.sync_copy(acc.at[pl.ds(row0, spt)], out_h.at[pl.ds(wid * spt, spt)])

    return pool(table, xp, dstb, zeros)


def _classify_tc(pooled, W, bT, *, B, L, PD, NCLS, BM):
    inv_l = 1.0 / L

    def body(p_ref, w_ref, b_ref, o_ref):
        o_ref[...] = (
            lax.dot_general(
                w_ref[...], p_ref[...], (((0,), (1,)), ((), ())),
                preferred_element_type=jnp.float32,
            )
            * inv_l
            + b_ref[...]
        )

    logits_t = pl.pallas_call(
        body,
        grid=(B // BM,),
        in_specs=[
            pl.BlockSpec((BM, PD), lambda i: (i, 0)),
            pl.BlockSpec((PD, NCLS), lambda i: (0, 0)),
            pl.BlockSpec((NCLS, 1), lambda i: (0, 0)),
        ],
        out_specs=pl.BlockSpec((NCLS, BM), lambda i: (0, i)),
        out_shape=jax.ShapeDtypeStruct((NCLS, B), jnp.float32),
    )(pooled, W, bT)
    return jnp.transpose(logits_t)


def kernel(x, table, W, b):
    B, L = x.shape
    VOCAB, DIM = table.shape
    NCLS = W.shape[1]
    PD = 128
    LP = 56
    spt = B // NW

    tablep = jnp.pad(table, ((0, 8), (0, PD - DIM)))
    xp = jnp.pad(x.astype(jnp.int32), ((0, 0), (0, LP - L)),
                 constant_values=VOCAB).reshape(B * LP)

    dstb_np = np.full((NS * spt, LP), NS * spt, dtype=np.int32)
    dstb_np[:, :L] = np.repeat(
        np.arange(NS * spt, dtype=np.int32), L).reshape(NS * spt, L)
    dstb = jnp.asarray(dstb_np.reshape(NS * spt * LP))
    zeros = jnp.asarray(np.zeros((spt, PD), np.float32))
    pooled = _pool_sc(tablep, xp, dstb, zeros, B=B, LP=LP, PD=PD)

    Wp = jnp.pad(W, ((0, PD - DIM), (0, 0)))
    return _classify_tc(pooled, Wp, b.reshape(NCLS, 1),
                        B=B, L=L, PD=PD, NCLS=NCLS, BM=1024)


# speedup vs baseline: 5.6867x; 5.6867x over previous
import functools

import jax
import jax.numpy as jnp
import numpy as np
from jax import lax
from jax.experimental import pallas as pl
from jax.experimental.pallas import tpu as pltpu
from jax.experimental.pallas import tpu_sc as plsc

NC = 2
NS = 16
NW = NC * NS

RR = 1
K = 4


def _pool_sc(table, x, dstb, zeros, *, B, L, DIM):
    spt = B // NW
    nj = spt // RR
    ng = nj // K
    grows = RR * L

    mesh = plsc.VectorSubcoreMesh(core_axis_name="c", subcore_axis_name="s")

    @functools.partial(
        pl.kernel,
        mesh=mesh,
        compiler_params=pltpu.CompilerParams(use_tc_tiling_on_sc=False),
        out_type=jax.ShapeDtypeStruct((B, DIM), jnp.float32),
        scratch_types=[
            pltpu.VMEM((spt, L), jnp.int32),
            pltpu.VMEM((spt, L), jnp.int32),
            pltpu.VMEM((K * RR * L, DIM), jnp.float32),
            pltpu.VMEM((K * RR * L, DIM), jnp.float32),
            pltpu.VMEM_SHARED((NS * (B // NW), DIM), jnp.float32),
            pltpu.SemaphoreType.DMA,
            pltpu.SemaphoreType.DMA,
            pltpu.SemaphoreType.DMA,
            pltpu.SemaphoreType.DMA,
        ],
    )
    def pool(table_h, x_h, dstb_h, zeros_h, out_h, idx_all, dst_all, rows0,
             rows1, acc, sem0, sem1, ssem0, ssem1):
        c = lax.axis_index("c")
        s = lax.axis_index("s")
        wid = c * NS + s
        row0 = s * spt

        pltpu.sync_copy(x_h.at[pl.ds(wid * spt, spt)], idx_all)
        pltpu.sync_copy(dstb_h.at[pl.ds(s * spt, spt)], dst_all)
        pltpu.sync_copy(zeros_h, acc.at[pl.ds(row0, spt)])

        rows = (rows0, rows1)
        sems = (sem0, sem1)
        ssems = (ssem0, ssem1)

        def fire(g, h):
            for k in range(K):
                j = g * K + k
                pltpu.async_copy(
                    table_h.at[idx_all.at[j]],
                    rows[h].at[pl.ds(k * grows, grows)],
                    sems[h],
                )

        def drain(sem, buf):
            pltpu.make_async_copy(
                table_h.at[pl.ds(0, K * grows)], buf, sem
            ).wait()

        fire(0, 0)

        def outer(gg, carry):
            for hh in range(2):
                g = gg * 2 + hh
                nh = (hh + 1) % 2

                @pl.when(g + 1 < ng)
                def _():
                    @pl.when(g >= 1)
                    def _():
                        drain(ssems[nh], rows[nh])
                    fire(g + 1, nh)

                drain(sems[hh], rows[hh])
                for k in range(K):
                    j = g * K + k
                    pltpu.async_copy(
                        rows[hh].at[pl.ds(k * grows, grows)],
                        acc.at[dst_all.at[j]],
                        ssems[hh],
                        add=True,
                    )
            return carry

        lax.fori_loop(0, ng // 2, outer, 0)
        drain(ssems[0], rows[0])
        drain(ssems[1], rows[1])

        pltpu.sync_copy(acc.at[pl.ds(row0, spt)], out_h.at[pl.ds(wid * spt, spt)])

    return pool(table, x, dstb, zeros)


def _classify_tc(pooled, W, bT, *, B, L, DIM, NCLS, BM):
    inv_l = 1.0 / L

    def body(p_ref, w_ref, b_ref, o_ref):
        o_ref[...] = (
            lax.dot_general(
                w_ref[...], p_ref[...], (((0,), (1,)), ((), ())),
                preferred_element_type=jnp.float32,
            )
            * inv_l
            + b_ref[...]
        )

    logits_t = pl.pallas_call(
        body,
        grid=(B // BM,),
        in_specs=[
            pl.BlockSpec((BM, DIM), lambda i: (i, 0)),
            pl.BlockSpec((DIM, NCLS), lambda i: (0, 0)),
            pl.BlockSpec((NCLS, 1), lambda i: (0, 0)),
        ],
        out_specs=pl.BlockSpec((NCLS, BM), lambda i: (0, i)),
        out_shape=jax.ShapeDtypeStruct((NCLS, B), jnp.float32),
    )(pooled, W, bT)
    return jnp.transpose(logits_t)


def kernel(x, table, W, b):
    B, L = x.shape
    DIM = table.shape[1]
    NCLS = W.shape[1]
    spt = B // NW

    dstb = jnp.asarray(
        np.repeat(np.arange(NS * spt, dtype=np.int32), L).reshape(NS * spt, L))
    zeros = jnp.asarray(np.zeros((spt, DIM), np.float32))
    pooled = _pool_sc(table, x.astype(jnp.int32), dstb, zeros, B=B, L=L, DIM=DIM)
    return _classify_tc(pooled, W, b.reshape(NCLS, 1),
                        B=B, L=L, DIM=DIM, NCLS=NCLS, BM=1024)
